# baseline (speedup 1.0000x reference)
import jax
import jax.numpy as jnp
from jax import lax
from jax.experimental import pallas as pl
from jax.experimental.pallas import tpu as pltpu
from jax.experimental.pallas import tpu_sc as plsc

EMBED_DIM = 11
BATCH = 16384


def _make_sc_gather(num_workers: int, w_per_w: int):
    mesh = plsc.VectorSubcoreMesh(core_axis_name="c", subcore_axis_name="s")

    @pl.kernel(
        out_type=jax.ShapeDtypeStruct((BATCH * EMBED_DIM,), jnp.float32),
        mesh=mesh,
        scratch_types=[
            pltpu.VMEM((w_per_w,), jnp.int32),
            pltpu.VMEM((w_per_w,), jnp.float32),
            pltpu.SemaphoreType.DMA,
        ],
        compiler_params=pltpu.CompilerParams(use_tc_tiling_on_sc=False),
    )
    def k(widx_hbm, table_hbm, out_hbm, idx_v, vals_v, sem):
        wid = lax.axis_index("s") * 2 + lax.axis_index("c")
        base = wid * w_per_w
        pltpu.sync_copy(widx_hbm.at[pl.ds(base, w_per_w)], idx_v)
        pltpu.async_copy(table_hbm.at[idx_v], vals_v, sem).wait()
        pltpu.sync_copy(vals_v, out_hbm.at[pl.ds(base, w_per_w)])

    return k


def kernel(device_num_tensor, table):
    info = plsc.get_sparse_core_info()
    num_workers = info.num_cores * info.num_subcores
    w_per_w = BATCH * EMBED_DIM // num_workers
    idx = device_num_tensor.astype(jnp.int32)
    widx = (idx[:, None] * EMBED_DIM + jnp.arange(EMBED_DIM, dtype=jnp.int32)).reshape(-1)
    flat = table.reshape(-1)
    out = _make_sc_gather(num_workers, w_per_w)(widx, flat)
    return out.reshape(BATCH, EMBED_DIM)


# speedup vs baseline: 1.0076x; 1.0076x over previous
import jax
import jax.numpy as jnp
from jax import lax
from jax.experimental import pallas as pl
from jax.experimental.pallas import tpu as pltpu
from jax.experimental.pallas import tpu_sc as plsc

EMBED_DIM = 11
PAD_DIM = 16
BATCH = 16384


def _make_sc_gather(num_workers: int, w_per_w: int):
    mesh = plsc.VectorSubcoreMesh(core_axis_name="c", subcore_axis_name="s")

    @pl.kernel(
        out_type=jax.ShapeDtypeStruct((BATCH * PAD_DIM,), jnp.float32),
        mesh=mesh,
        scratch_types=[
            pltpu.VMEM((w_per_w,), jnp.int32),
            pltpu.VMEM((w_per_w,), jnp.float32),
            pltpu.SemaphoreType.DMA,
        ],
        compiler_params=pltpu.CompilerParams(use_tc_tiling_on_sc=False),
    )
    def k(widx_hbm, table_hbm, out_hbm, idx_v, vals_v, sem):
        wid = lax.axis_index("s") * 2 + lax.axis_index("c")
        base = wid * w_per_w
        pltpu.sync_copy(widx_hbm.at[pl.ds(base, w_per_w)], idx_v)
        pltpu.async_copy(table_hbm.at[idx_v], vals_v, sem).wait()
        pltpu.sync_copy(vals_v, out_hbm.at[pl.ds(base, w_per_w)])

    return k


def kernel(device_num_tensor, table):
    info = plsc.get_sparse_core_info()
    num_workers = info.num_cores * info.num_subcores
    w_per_w = BATCH * PAD_DIM // num_workers
    idx = device_num_tensor.astype(jnp.int32)
    col = jnp.minimum(jnp.arange(PAD_DIM, dtype=jnp.int32), EMBED_DIM - 1)
    widx = (idx[:, None] * EMBED_DIM + col).reshape(-1)
    flat = table.reshape(-1)
    out = _make_sc_gather(num_workers, w_per_w)(widx, flat)
    return out.reshape(BATCH, PAD_DIM)[:, :EMBED_DIM]
